# Initial kernel scaffold; baseline (speedup 1.0000x reference)
import jax
import jax.numpy as jnp
from jax.experimental import pallas as pl


def kernel(feat, edge_index, in_deg, W, b, ln_gamma, ln_beta):
    raise NotImplementedError("write your pallas kernel here")


# speedup vs baseline: 7.6152x; 7.6152x over previous
import functools

import jax
import jax.numpy as jnp
from jax import lax
from jax.experimental import pallas as pl
from jax.experimental.pallas import tpu as pltpu
from jax.experimental.pallas import tpu_sc as plsc

N = 10000
E = 320000
D = 128
_EPS = 1e-5

_LN_BLOCK = 2000


def _ln_relu_body(feat_ref, g_ref, b_ref, out_ref):
    x = feat_ref[...]
    mean = jnp.mean(x, axis=-1, keepdims=True)
    xc = x - mean
    var = jnp.mean(xc * xc, axis=-1, keepdims=True)
    h = xc * lax.rsqrt(var + _EPS) * g_ref[...] + b_ref[...]
    out_ref[...] = jnp.maximum(h, 0.0)


def _ln_relu(feat, gamma2d, beta2d):
    n, d = feat.shape
    return pl.pallas_call(
        _ln_relu_body,
        grid=(n // _LN_BLOCK,),
        in_specs=[
            pl.BlockSpec((_LN_BLOCK, d), lambda i: (i, 0)),
            pl.BlockSpec((1, d), lambda i: (0, 0)),
            pl.BlockSpec((1, d), lambda i: (0, 0)),
        ],
        out_specs=pl.BlockSpec((_LN_BLOCK, d), lambda i: (i, 0)),
        out_shape=jax.ShapeDtypeStruct((n, d), jnp.float32),
    )(feat, gamma2d, beta2d)


_NC = 2
_NS = 16
_NW = _NC * _NS
_EPT = E // _NW
_CH = 80
_NCHUNK = _EPT // _CH
_NPAIR = (_NCHUNK - 1) // 2
_RPT = 632
_NPAD = _RPT * _NS


def _make_sc_aggregate():
    mesh = plsc.VectorSubcoreMesh(core_axis_name="c", subcore_axis_name="s")

    @functools.partial(
        pl.kernel,
        out_type=jax.ShapeDtypeStruct((_NC, _NPAD, D), jnp.float32),
        mesh=mesh,
        scratch_types=[
            pltpu.VMEM((_CH,), jnp.int32),
            pltpu.VMEM((_CH,), jnp.int32),
            pltpu.VMEM((_CH,), jnp.int32),
            pltpu.VMEM((_CH,), jnp.int32),
            pltpu.VMEM((_CH, D), jnp.float32),
            pltpu.VMEM((_CH, D), jnp.float32),
            pltpu.VMEM_SHARED((_NPAD, D), jnp.float32),
            pltpu.SemaphoreType.DMA,
            pltpu.SemaphoreType.DMA,
        ],
    )
    def sc_aggregate(h_hbm, src_hbm, dst_hbm, zero_hbm, out_hbm,
                     src_a, src_b, dst_a, dst_b, msg_a, msg_b, agg,
                     sem_a, sem_b):
        c = lax.axis_index("c")
        s = lax.axis_index("s")
        ebase = (c * _NS + s) * _EPT

        r0 = s * _RPT
        pltpu.sync_copy(zero_hbm.at[pl.ds(r0, _RPT)], agg.at[pl.ds(r0, _RPT)])
        plsc.subcore_barrier()

        def load_idx(ci, srcb, dstb):
            base = pl.multiple_of(ebase + ci * _CH, 8)
            pltpu.sync_copy(src_hbm.at[pl.ds(base, _CH)], srcb)
            pltpu.sync_copy(dst_hbm.at[pl.ds(base, _CH)], dstb)

        load_idx(0, src_a, dst_a)
        pltpu.async_copy(h_hbm.at[src_a], msg_a, sem_a)
        load_idx(1, src_b, dst_b)
        pltpu.async_copy(h_hbm.at[src_b], msg_b, sem_b)

        def pair(g, carry):
            pltpu.make_async_copy(h_hbm.at[src_a], msg_a, sem_a).wait()
            pltpu.sync_copy(msg_a, agg.at[dst_a], add=True)
            load_idx(2 * g + 2, src_a, dst_a)
            pltpu.async_copy(h_hbm.at[src_a], msg_a, sem_a)
            pltpu.make_async_copy(h_hbm.at[src_b], msg_b, sem_b).wait()
            pltpu.sync_copy(msg_b, agg.at[dst_b], add=True)

            @pl.when(g + 1 < _NPAIR)
            def _():
                load_idx(2 * g + 3, src_b, dst_b)
                pltpu.async_copy(h_hbm.at[src_b], msg_b, sem_b)

            return carry

        lax.fori_loop(0, _NPAIR, pair, 0)

        pltpu.make_async_copy(h_hbm.at[src_a], msg_a, sem_a).wait()
        pltpu.sync_copy(msg_a, agg.at[dst_a], add=True)

        plsc.subcore_barrier()
        pltpu.sync_copy(agg.at[pl.ds(r0, _RPT)], out_hbm.at[c, pl.ds(r0, _RPT)])

    return sc_aggregate


_sc_aggregate = _make_sc_aggregate()


_MM_BLOCK = 1000


def _mm_body(p_ref, feat_ref, deg_ref, w_ref, b_ref, out_ref):
    agg = p_ref[0] + p_ref[1]
    y = jnp.dot(agg, w_ref[...], preferred_element_type=jnp.float32)
    y = y / deg_ref[...]
    out_ref[...] = feat_ref[...] + y + b_ref[...]


def _mm_residual(partial, feat, deg2d, W, bias2d):
    n, d = feat.shape
    return pl.pallas_call(
        _mm_body,
        grid=(n // _MM_BLOCK,),
        in_specs=[
            pl.BlockSpec((_NC, _MM_BLOCK, d), lambda i: (0, i, 0)),
            pl.BlockSpec((_MM_BLOCK, d), lambda i: (i, 0)),
            pl.BlockSpec((_MM_BLOCK, 1), lambda i: (i, 0)),
            pl.BlockSpec((d, d), lambda i: (0, 0)),
            pl.BlockSpec((1, d), lambda i: (0, 0)),
        ],
        out_specs=pl.BlockSpec((_MM_BLOCK, d), lambda i: (i, 0)),
        out_shape=jax.ShapeDtypeStruct((n, d), jnp.float32),
    )(partial, feat, deg2d, W, bias2d)


def kernel(feat, edge_index, in_deg, W, b, ln_gamma, ln_beta):
    n, d = feat.shape
    h = _ln_relu(feat, ln_gamma.reshape(1, d), ln_beta.reshape(1, d))
    zeros = jnp.zeros((_NPAD, d), jnp.float32)
    partial = _sc_aggregate(h, edge_index[0], edge_index[1], zeros)
    return _mm_residual(partial, feat, in_deg.reshape(n, 1), W,
                        b.reshape(1, d))


# speedup vs baseline: 9.3685x; 1.2302x over previous
import functools

import jax
import jax.numpy as jnp
from jax import lax
from jax.experimental import pallas as pl
from jax.experimental.pallas import tpu as pltpu
from jax.experimental.pallas import tpu_sc as plsc

N = 10000
E = 320000
D = 128
_EPS = 1e-5

_LN_BLOCK = 2000


def _ln_relu_body(feat_ref, g_ref, b_ref, out_ref):
    x = feat_ref[...]
    mean = jnp.mean(x, axis=-1, keepdims=True)
    xc = x - mean
    var = jnp.mean(xc * xc, axis=-1, keepdims=True)
    h = xc * lax.rsqrt(var + _EPS) * g_ref[...] + b_ref[...]
    out_ref[...] = jnp.maximum(h, 0.0)


def _ln_relu(feat, gamma2d, beta2d):
    n, d = feat.shape
    return pl.pallas_call(
        _ln_relu_body,
        grid=(n // _LN_BLOCK,),
        in_specs=[
            pl.BlockSpec((_LN_BLOCK, d), lambda i: (i, 0)),
            pl.BlockSpec((1, d), lambda i: (0, 0)),
            pl.BlockSpec((1, d), lambda i: (0, 0)),
        ],
        out_specs=pl.BlockSpec((_LN_BLOCK, d), lambda i: (i, 0)),
        out_shape=jax.ShapeDtypeStruct((n, d), jnp.float32),
    )(feat, gamma2d, beta2d)


_NC = 2
_NS = 16
_NW = _NC * _NS
_EPT = E // _NW
_CH = 40
_NCHUNK = _EPT // _CH
_MRING = 4
_IRING = 8
_UNROLL = 8
_NG = 32
_RPT = 632
_NPAD = _RPT * _NS


def _make_sc_aggregate():
    mesh = plsc.VectorSubcoreMesh(core_axis_name="c", subcore_axis_name="s")

    @functools.partial(
        pl.kernel,
        out_type=jax.ShapeDtypeStruct((_NC, _NPAD, D), jnp.float32),
        mesh=mesh,
        scratch_types=[
            [pltpu.VMEM((2, _CH), jnp.int32) for _ in range(_IRING)],
            [pltpu.VMEM((_CH, D), jnp.float32) for _ in range(_MRING)],
            pltpu.VMEM_SHARED((_NPAD, D), jnp.float32),
            [pltpu.SemaphoreType.DMA for _ in range(_IRING)],
            [pltpu.SemaphoreType.DMA for _ in range(_MRING)],
            [pltpu.SemaphoreType.DMA for _ in range(_MRING)],
        ],
    )
    def sc_aggregate(h_hbm, idx_hbm, zero_hbm, out_hbm,
                     idx_v, msg_v, agg, isem, gsem, ssem):
        c = lax.axis_index("c")
        s = lax.axis_index("s")
        w = c * _NS + s

        r0 = s * _RPT
        pltpu.sync_copy(zero_hbm.at[pl.ds(r0, _RPT)], agg.at[pl.ds(r0, _RPT)])
        plsc.subcore_barrier()

        def idx_load(ci, j):
            pltpu.async_copy(idx_hbm.at[w, ci], idx_v[j], isem[j])

        def idx_wait(ci, j):
            pltpu.make_async_copy(idx_hbm.at[w, ci], idx_v[j], isem[j]).wait()

        def gather(j8, j4):
            pltpu.async_copy(h_hbm.at[idx_v[j8].at[0]], msg_v[j4], gsem[j4])

        def gather_wait(j8, j4):
            pltpu.make_async_copy(h_hbm.at[idx_v[j8].at[0]], msg_v[j4],
                                  gsem[j4]).wait()

        def scatter(j8, j4):
            pltpu.async_copy(msg_v[j4], agg.at[idx_v[j8].at[1]], ssem[j4],
                             add=True)

        def scatter_wait(j8, j4):
            pltpu.make_async_copy(msg_v[j4], agg.at[idx_v[j8].at[1]],
                                  ssem[j4]).wait()

        for j in range(6):
            idx_load(j, j)
        idx_wait(0, 0)
        gather(0, 0)
        idx_wait(1, 1)
        gather(1, 1)

        def step(g, carry):
            for j in range(_UNROLL):
                ci = g * _UNROLL + j
                j4 = j % _MRING
                j8 = j % _IRING

                @pl.when(jnp.logical_and(ci >= 2, ci < _NCHUNK + 2))
                def _():
                    scatter_wait((j8 - 2) % _IRING, (j4 - 2) % _MRING)

                @pl.when(ci + 6 < _NCHUNK)
                def _():
                    idx_load(ci + 6, (j8 + 6) % _IRING)

                @pl.when(ci + 2 < _NCHUNK)
                def _():
                    idx_wait(ci + 2, (j8 + 2) % _IRING)
                    gather((j8 + 2) % _IRING, (j4 + 2) % _MRING)

                @pl.when(ci < _NCHUNK)
                def _():
                    gather_wait(j8, j4)
                    scatter(j8, j4)

            return carry

        lax.fori_loop(0, _NG, step, 0)

        plsc.subcore_barrier()
        pltpu.sync_copy(agg.at[pl.ds(r0, _RPT)], out_hbm.at[c, pl.ds(r0, _RPT)])

    return sc_aggregate


_sc_aggregate = _make_sc_aggregate()


_MM_BLOCK = 1000


def _mm_body(p_ref, feat_ref, deg_ref, w_ref, b_ref, out_ref):
    agg = p_ref[0] + p_ref[1]
    y = jnp.dot(agg, w_ref[...], preferred_element_type=jnp.float32)
    y = y / deg_ref[...]
    out_ref[...] = feat_ref[...] + y + b_ref[...]


def _mm_residual(partial, feat, deg2d, W, bias2d):
    n, d = feat.shape
    return pl.pallas_call(
        _mm_body,
        grid=(n // _MM_BLOCK,),
        in_specs=[
            pl.BlockSpec((_NC, _MM_BLOCK, d), lambda i: (0, i, 0)),
            pl.BlockSpec((_MM_BLOCK, d), lambda i: (i, 0)),
            pl.BlockSpec((_MM_BLOCK, 1), lambda i: (i, 0)),
            pl.BlockSpec((d, d), lambda i: (0, 0)),
            pl.BlockSpec((1, d), lambda i: (0, 0)),
        ],
        out_specs=pl.BlockSpec((_MM_BLOCK, d), lambda i: (i, 0)),
        out_shape=jax.ShapeDtypeStruct((n, d), jnp.float32),
    )(partial, feat, deg2d, W, bias2d)


def kernel(feat, edge_index, in_deg, W, b, ln_gamma, ln_beta):
    n, d = feat.shape
    h = _ln_relu(feat, ln_gamma.reshape(1, d), ln_beta.reshape(1, d))
    zeros = jnp.zeros((_NPAD, d), jnp.float32)
    idx = jnp.stack(
        [edge_index[0].reshape(_NW, _NCHUNK, _CH),
         edge_index[1].reshape(_NW, _NCHUNK, _CH)], axis=2)
    partial = _sc_aggregate(h, idx, zeros)
    return _mm_residual(partial, feat, in_deg.reshape(n, 1), W,
                        b.reshape(1, d))
